# baseline (speedup 1.0000x reference)
import functools

import jax
import jax.numpy as jnp
from jax import lax
from jax.experimental import pallas as pl
from jax.experimental.pallas import tpu as pltpu
from jax.experimental.pallas import tpu_sc as plsc

NC = 2
NS = 16
K = 128
W = 128
RB = 1000
SROWS = 640
NPAD = NS * SROWS


def _agg_mesh():
    return plsc.VectorSubcoreMesh(
        core_axis_name="c", subcore_axis_name="s", num_cores=NC, num_subcores=NS)


def _make_deg(e_rows):
    rpt = e_rows // (NC * NS)

    @functools.partial(
        pl.kernel,
        out_type=jax.ShapeDtypeStruct((NC, NPAD, W), jnp.float32),
        mesh=_agg_mesh(),
        scratch_types=[
            pltpu.VMEM((rpt, K), jnp.int32),
            pltpu.VMEM((K, W), jnp.float32),
            pltpu.VMEM_SHARED((NPAD, W), jnp.float32),
        ],
    )
    def deg_kernel(dst_hbm, ones_hbm, zeros_hbm, out_hbm, dstb, onesb, acc):
        cid = lax.axis_index("c")
        sid = lax.axis_index("s")
        wid = cid * NS + sid
        pltpu.sync_copy(ones_hbm, onesb)
        pltpu.sync_copy(zeros_hbm, acc.at[pl.ds(sid * SROWS, SROWS)])
        plsc.subcore_barrier()
        pltpu.sync_copy(dst_hbm.at[pl.ds(wid * rpt, rpt)], dstb)

        def step(j, carry):
            pltpu.sync_copy(onesb, acc.at[dstb.at[j]], add=True)
            return carry

        lax.fori_loop(0, rpt, step, 0)
        plsc.subcore_barrier()
        pltpu.sync_copy(acc.at[pl.ds(sid * SROWS, SROWS)],
                        out_hbm.at[cid, pl.ds(sid * SROWS, SROWS)])

    return deg_kernel


def _make_agg(e_rows, n_chunks):
    split32 = n_chunks == 1
    rpt = e_rows // (NC * NS) if split32 else e_rows // NS
    n_planes = NC if split32 else n_chunks
    IB = 16
    assert rpt % IB == 0

    @functools.partial(
        pl.kernel,
        out_type=jax.ShapeDtypeStruct((n_planes, NPAD, W), jnp.float32),
        mesh=_agg_mesh(),
        scratch_types=[
            pltpu.VMEM((IB, K), jnp.int32),
            pltpu.VMEM((IB, K), jnp.int32),
            pltpu.VMEM((K, W), jnp.float32),
            pltpu.VMEM_SHARED((NPAD, W), jnp.float32),
            pltpu.SemaphoreType.DMA,
        ],
    )
    def agg_kernel(src_hbm, dst_hbm, zeros_hbm, *rest):
        tables = rest[:n_chunks]
        out_hbm = rest[n_chunks]
        srcb, dstb, rowsb, acc, sem = rest[n_chunks + 1:]
        cid = lax.axis_index("c")
        sid = lax.axis_index("s")

        def do_round(table, plane):
            pltpu.sync_copy(zeros_hbm, acc.at[pl.ds(sid * SROWS, SROWS)])
            plsc.subcore_barrier()
            base = ((cid * NS + sid) if split32 else sid) * rpt

            def block(b, carry):
                pltpu.sync_copy(src_hbm.at[pl.ds(base + b * IB, IB)], srcb)
                pltpu.sync_copy(dst_hbm.at[pl.ds(base + b * IB, IB)], dstb)

                def step(j, carry2):
                    pltpu.async_copy(table.at[srcb.at[j]], rowsb, sem).wait()
                    pltpu.sync_copy(rowsb, acc.at[dstb.at[j]], add=True)
                    return carry2

                lax.fori_loop(0, IB, step, 0)
                return carry

            lax.fori_loop(0, rpt // IB, block, 0)
            plsc.subcore_barrier()
            pltpu.sync_copy(acc.at[pl.ds(sid * SROWS, SROWS)],
                            out_hbm.at[plane, pl.ds(sid * SROWS, SROWS)])

        if split32:
            do_round(tables[0], cid)
        else:
            rounds = n_chunks // NC
            for r in range(rounds):
                for core in range(NC):
                    chunk = core * rounds + r

                    @pl.when(cid == core)
                    def _(chunk=chunk):
                        do_round(tables[chunk], chunk)

    return agg_kernel


def _dotT(a, b_ref):
    return lax.dot_general(a, b_ref[...], (((1,), (1,)), ((), ())),
                           preferred_element_type=jnp.float32)


def _pad_chunks(xs, d, n_chunks):
    outs = []
    for c in range(n_chunks):
        piece = xs[:, c * W:min((c + 1) * W, d)]
        if piece.shape[1] < W:
            piece = jnp.concatenate(
                [piece, jnp.zeros((piece.shape[0], W - piece.shape[1]),
                                  jnp.float32)], axis=1)
        outs.append(piece)
    return outs


def _t1_first(n, d_in, d_out, n_chunks):
    nb = n // RB

    def body(x_ref, degp_ref, w_ref, *out_refs):
        xs_refs = out_refs[:n_chunks]
        dinv_ref = out_refs[n_chunks]
        deg = degp_ref[0, :, 0:1] + degp_ref[1, :, 0:1] + 1.0
        dinv = 1.0 / jnp.sqrt(deg)
        xs = _dotT(x_ref[...], w_ref) * dinv
        for oref, piece in zip(xs_refs, _pad_chunks(xs, d_out, n_chunks)):
            oref[...] = piece
        dinv_ref[...] = dinv

    return pl.pallas_call(
        body,
        grid=(nb,),
        in_specs=[
            pl.BlockSpec((RB, d_in), lambda r: (r, 0)),
            pl.BlockSpec((NC, RB, W), lambda r: (0, r, 0)),
            pl.BlockSpec((d_out, d_in), lambda r: (0, 0)),
        ],
        out_specs=[pl.BlockSpec((RB, W), lambda r: (r, 0))
                   for _ in range(n_chunks)]
        + [pl.BlockSpec((RB, 1), lambda r: (r, 0))],
        out_shape=[jax.ShapeDtypeStruct((n, W), jnp.float32)
                   for _ in range(n_chunks)]
        + [jax.ShapeDtypeStruct((n, 1), jnp.float32)],
    )


def _t3(n, d):
    nb = n // RB

    def body(z_ref, st_ref, ss_ref):
        mean = st_ref[0:1, :] * (1.0 / n)
        dz = z_ref[...] - mean
        ps = jnp.concatenate(
            [jnp.sum(dz * dz, axis=0, keepdims=True),
             jnp.zeros((7, d), jnp.float32)], axis=0)
        r = pl.program_id(0)

        @pl.when(r == 0)
        def _():
            ss_ref[...] = ps

        @pl.when(r > 0)
        def _():
            ss_ref[...] = ss_ref[...] + ps

    return pl.pallas_call(
        body,
        grid=(nb,),
        in_specs=[
            pl.BlockSpec((RB, d), lambda r: (r, 0)),
            pl.BlockSpec((8, d), lambda r: (0, 0)),
        ],
        out_specs=pl.BlockSpec((8, d), lambda r: (0, 0)),
        out_shape=jax.ShapeDtypeStruct((8, d), jnp.float32),
    )


def _t1(n, d_in, d_out, n_chunks):
    nb = n // RB

    def body(z_ref, st_ref, ss_ref, g_ref, bt_ref, w_ref, dinv_ref, *out_refs):
        mean = st_ref[0:1, :] * (1.0 / n)
        var = ss_ref[0:1, :] * (1.0 / n)
        inv = 1.0 / jnp.sqrt(var + 1e-5)
        h = (z_ref[...] - mean) * (inv * g_ref[...]) + bt_ref[...]
        h = jnp.where(h >= 0, h, 0.01 * h)
        xs = _dotT(h, w_ref) * dinv_ref[...]
        for oref, piece in zip(out_refs, _pad_chunks(xs, d_out, n_chunks)):
            oref[...] = piece

    return pl.pallas_call(
        body,
        grid=(nb,),
        in_specs=[
            pl.BlockSpec((RB, d_in), lambda r: (r, 0)),
            pl.BlockSpec((8, d_in), lambda r: (0, 0)),
            pl.BlockSpec((8, d_in), lambda r: (0, 0)),
            pl.BlockSpec((1, d_in), lambda r: (0, 0)),
            pl.BlockSpec((1, d_in), lambda r: (0, 0)),
            pl.BlockSpec((d_out, d_in), lambda r: (0, 0)),
            pl.BlockSpec((RB, 1), lambda r: (r, 0)),
        ],
        out_specs=[pl.BlockSpec((RB, W), lambda r: (r, 0))
                   for _ in range(n_chunks)],
        out_shape=[jax.ShapeDtypeStruct((n, W), jnp.float32)
                   for _ in range(n_chunks)],
    )


def _t2(n, d, n_chunks, n_planes):
    nb = n // RB
    split = n_chunks == 1

    def body(accp_ref, *rest):
        xs_refs = rest[:n_chunks]
        dinv_ref, b_ref, z_ref, st_ref = rest[n_chunks:]
        parts = []
        for c in range(n_chunks):
            a = (accp_ref[0] + accp_ref[1]) if split else accp_ref[c]
            wc = min((c + 1) * W, d) - c * W
            parts.append((a + xs_refs[c][...])[:, :wc])
        agg = jnp.concatenate(parts, axis=1) if n_chunks > 1 else parts[0]
        z = agg * dinv_ref[...] + b_ref[...]
        z_ref[...] = z
        ps = jnp.concatenate(
            [jnp.sum(z, axis=0, keepdims=True),
             jnp.sum(z * z, axis=0, keepdims=True),
             jnp.zeros((6, d), jnp.float32)], axis=0)
        r = pl.program_id(0)

        @pl.when(r == 0)
        def _():
            st_ref[...] = ps

        @pl.when(r > 0)
        def _():
            st_ref[...] = st_ref[...] + ps

    return pl.pallas_call(
        body,
        grid=(nb,),
        in_specs=[pl.BlockSpec((n_planes, RB, W), lambda r: (0, r, 0))]
        + [pl.BlockSpec((RB, W), lambda r: (r, 0)) for _ in range(n_chunks)]
        + [
            pl.BlockSpec((RB, 1), lambda r: (r, 0)),
            pl.BlockSpec((1, d), lambda r: (0, 0)),
        ],
        out_specs=[
            pl.BlockSpec((RB, d), lambda r: (r, 0)),
            pl.BlockSpec((8, d), lambda r: (0, 0)),
        ],
        out_shape=[
            jax.ShapeDtypeStruct((n, d), jnp.float32),
            jax.ShapeDtypeStruct((8, d), jnp.float32),
        ],
    )


def _head(n, d, d_out):
    nb = n // RB

    def body(z_ref, st_ref, ss_ref, g_ref, bt_ref, lw_ref, lb_ref, o_ref):
        mean = st_ref[0:1, :] * (1.0 / n)
        var = ss_ref[0:1, :] * (1.0 / n)
        inv = 1.0 / jnp.sqrt(var + 1e-5)
        h = (z_ref[...] - mean) * (inv * g_ref[...]) + bt_ref[...]
        h = jnp.where(h >= 0, h, 0.01 * h)
        t = jnp.tanh(_dotT(h, lw_ref) + lb_ref[...])
        s = jnp.sum(t * t, axis=1, keepdims=True)
        o_ref[...] = t / (jnp.sqrt(s) + 1e-12)

    return pl.pallas_call(
        body,
        grid=(nb,),
        in_specs=[
            pl.BlockSpec((RB, d), lambda r: (r, 0)),
            pl.BlockSpec((8, d), lambda r: (0, 0)),
            pl.BlockSpec((8, d), lambda r: (0, 0)),
            pl.BlockSpec((1, d), lambda r: (0, 0)),
            pl.BlockSpec((1, d), lambda r: (0, 0)),
            pl.BlockSpec((d_out, d), lambda r: (0, 0)),
            pl.BlockSpec((1, d_out), lambda r: (0, 0)),
        ],
        out_specs=pl.BlockSpec((RB, d_out), lambda r: (r, 0)),
        out_shape=jax.ShapeDtypeStruct((n, d_out), jnp.float32),
    )


def kernel(x, edge_index, Ws, bs, gammas, betas, linW, linb):
    n = x.shape[0]
    e = edge_index.shape[1]
    n_layers = len(Ws)
    src = edge_index[0]
    dst = edge_index[1]

    e_rows = -(-e // (K * 64)) * 64
    ep = e_rows * K
    pad = ep - e
    pad_idx = jnp.arange(pad, dtype=jnp.int32)
    src_p = jnp.concatenate([src, (pad_idx * 97) % n])
    dst_p = jnp.concatenate([dst, n + (pad_idx % 64)])
    src2 = src_p.reshape(e_rows, K)
    dst2 = dst_p.reshape(e_rows, K)

    zeros_w = jnp.zeros((SROWS, W), jnp.float32)

    deg_p = _make_deg(e_rows)(
        dst2, jnp.ones((K, W), jnp.float32), zeros_w)

    dinv = None
    z = None
    stats = None
    ssq = None
    for i in range(n_layers):
        d_in = Ws[i].shape[1]
        d = Ws[i].shape[0]
        n_chunks = -(-d // W)
        n_planes = NC if n_chunks == 1 else n_chunks

        if i == 0:
            *xs_chunks, dinv = _t1_first(n, d_in, d, n_chunks)(
                x, deg_p, Ws[i])
        else:
            xs_chunks = list(_t1(n, d_in, d, n_chunks)(
                z, stats, ssq, gammas[i - 1].reshape(1, -1),
                betas[i - 1].reshape(1, -1), Ws[i], dinv))

        acc_p = _make_agg(e_rows, n_chunks)(
            src2, dst2, zeros_w, *xs_chunks)
        z, stats = _t2(n, d, n_chunks, n_planes)(
            acc_p, *xs_chunks, dinv, bs[i].reshape(1, -1))
        ssq = _t3(n, d)(z, stats)

    out = _head(n, Ws[-1].shape[0], linW.shape[0])(
        z, stats, ssq, gammas[-1].reshape(1, -1), betas[-1].reshape(1, -1),
        linW, linb.reshape(1, -1))
    return out


# speedup vs baseline: 1.2726x; 1.2726x over previous
import functools

import jax
import jax.numpy as jnp
from jax import lax
from jax.experimental import pallas as pl
from jax.experimental.pallas import tpu as pltpu
from jax.experimental.pallas import tpu_sc as plsc

NC = 2
NS = 16
K = 128
W = 128
RB = 1000
SROWS = 640
NPAD = NS * SROWS


def _agg_mesh():
    return plsc.VectorSubcoreMesh(
        core_axis_name="c", subcore_axis_name="s", num_cores=NC, num_subcores=NS)


def _make_deg(e_rows):
    rpt = e_rows // (NC * NS)

    @functools.partial(
        pl.kernel,
        out_type=jax.ShapeDtypeStruct((NC, NPAD, W), jnp.float32),
        mesh=_agg_mesh(),
        scratch_types=[
            pltpu.VMEM((rpt, K), jnp.int32),
            pltpu.VMEM((K, W), jnp.float32),
            pltpu.VMEM_SHARED((NPAD, W), jnp.float32),
        ],
    )
    def deg_kernel(dst_hbm, ones_hbm, zeros_hbm, out_hbm, dstb, onesb, acc):
        cid = lax.axis_index("c")
        sid = lax.axis_index("s")
        wid = cid * NS + sid
        pltpu.sync_copy(ones_hbm, onesb)
        pltpu.sync_copy(zeros_hbm, acc.at[pl.ds(sid * SROWS, SROWS)])
        plsc.subcore_barrier()
        pltpu.sync_copy(dst_hbm.at[pl.ds(wid * rpt, rpt)], dstb)

        def step(j, carry):
            pltpu.sync_copy(onesb, acc.at[dstb.at[j]], add=True)
            return carry

        lax.fori_loop(0, rpt, step, 0)
        plsc.subcore_barrier()
        pltpu.sync_copy(acc.at[pl.ds(sid * SROWS, SROWS)],
                        out_hbm.at[cid, pl.ds(sid * SROWS, SROWS)])

    return deg_kernel


def _make_agg(e_rows, n_chunks):
    split32 = n_chunks == 1
    rpt = e_rows // (NC * NS) if split32 else e_rows // NS
    n_planes = NC if split32 else n_chunks
    IB = 16
    assert rpt % IB == 0

    @functools.partial(
        pl.kernel,
        out_type=jax.ShapeDtypeStruct((n_planes, NPAD, W), jnp.float32),
        mesh=_agg_mesh(),
        scratch_types=[
            pltpu.VMEM((IB, K), jnp.int32),
            pltpu.VMEM((IB, K), jnp.int32),
            pltpu.VMEM((K, W), jnp.float32),
            pltpu.VMEM((K, W), jnp.float32),
            pltpu.VMEM_SHARED((NPAD, W), jnp.float32),
            pltpu.SemaphoreType.DMA,
        ],
    )
    def agg_kernel(src_hbm, dst_hbm, zeros_hbm, *rest):
        tables = rest[:n_chunks]
        out_hbm = rest[n_chunks]
        srcb, dstb, rows0, rows1, acc, sem = rest[n_chunks + 1:]
        cid = lax.axis_index("c")
        sid = lax.axis_index("s")

        def do_round(table, plane):
            pltpu.sync_copy(zeros_hbm, acc.at[pl.ds(sid * SROWS, SROWS)])
            plsc.subcore_barrier()
            base = ((cid * NS + sid) if split32 else sid) * rpt

            def block(b, carry):
                pltpu.sync_copy(src_hbm.at[pl.ds(base + b * IB, IB)], srcb)
                pltpu.sync_copy(dst_hbm.at[pl.ds(base + b * IB, IB)], dstb)
                pltpu.make_async_copy(table.at[srcb.at[0]], rows0, sem).start()

                def pair(t, carry2):
                    j0 = 2 * t
                    j1 = 2 * t + 1
                    pltpu.make_async_copy(
                        table.at[srcb.at[j0]], rows0, sem).wait()
                    pltpu.make_async_copy(
                        table.at[srcb.at[j1]], rows1, sem).start()
                    pltpu.sync_copy(rows0, acc.at[dstb.at[j0]], add=True)
                    pltpu.make_async_copy(
                        table.at[srcb.at[j1]], rows1, sem).wait()

                    @pl.when(j1 + 1 < IB)
                    def _():
                        pltpu.make_async_copy(
                            table.at[srcb.at[j1 + 1]], rows0, sem).start()

                    pltpu.sync_copy(rows1, acc.at[dstb.at[j1]], add=True)
                    return carry2

                lax.fori_loop(0, IB // 2, pair, 0)
                return carry

            lax.fori_loop(0, rpt // IB, block, 0)
            plsc.subcore_barrier()
            pltpu.sync_copy(acc.at[pl.ds(sid * SROWS, SROWS)],
                            out_hbm.at[plane, pl.ds(sid * SROWS, SROWS)])

        if split32:
            do_round(tables[0], cid)
        else:
            rounds = n_chunks // NC
            for r in range(rounds):
                for core in range(NC):
                    chunk = core * rounds + r

                    @pl.when(cid == core)
                    def _(chunk=chunk):
                        do_round(tables[chunk], chunk)

    return agg_kernel


def _dotT(a, b_ref):
    return lax.dot_general(a, b_ref[...], (((1,), (1,)), ((), ())),
                           preferred_element_type=jnp.float32)


def _pad_chunks(xs, d, n_chunks):
    outs = []
    for c in range(n_chunks):
        piece = xs[:, c * W:min((c + 1) * W, d)]
        if piece.shape[1] < W:
            piece = jnp.concatenate(
                [piece, jnp.zeros((piece.shape[0], W - piece.shape[1]),
                                  jnp.float32)], axis=1)
        outs.append(piece)
    return outs


def _t1_first(n, d_in, d_out, n_chunks):
    nb = n // RB

    def body(x_ref, degp_ref, w_ref, *out_refs):
        xs_refs = out_refs[:n_chunks]
        dinv_ref = out_refs[n_chunks]
        deg = degp_ref[0, :, 0:1] + degp_ref[1, :, 0:1] + 1.0
        dinv = 1.0 / jnp.sqrt(deg)
        xs = _dotT(x_ref[...], w_ref) * dinv
        for oref, piece in zip(xs_refs, _pad_chunks(xs, d_out, n_chunks)):
            oref[...] = piece
        dinv_ref[...] = dinv

    return pl.pallas_call(
        body,
        grid=(nb,),
        in_specs=[
            pl.BlockSpec((RB, d_in), lambda r: (r, 0)),
            pl.BlockSpec((NC, RB, W), lambda r: (0, r, 0)),
            pl.BlockSpec((d_out, d_in), lambda r: (0, 0)),
        ],
        out_specs=[pl.BlockSpec((RB, W), lambda r: (r, 0))
                   for _ in range(n_chunks)]
        + [pl.BlockSpec((RB, 1), lambda r: (r, 0))],
        out_shape=[jax.ShapeDtypeStruct((n, W), jnp.float32)
                   for _ in range(n_chunks)]
        + [jax.ShapeDtypeStruct((n, 1), jnp.float32)],
    )


def _t3(n, d):
    nb = n // RB

    def body(z_ref, st_ref, ss_ref):
        mean = st_ref[0:1, :] * (1.0 / n)
        dz = z_ref[...] - mean
        ps = jnp.concatenate(
            [jnp.sum(dz * dz, axis=0, keepdims=True),
             jnp.zeros((7, d), jnp.float32)], axis=0)
        r = pl.program_id(0)

        @pl.when(r == 0)
        def _():
            ss_ref[...] = ps

        @pl.when(r > 0)
        def _():
            ss_ref[...] = ss_ref[...] + ps

    return pl.pallas_call(
        body,
        grid=(nb,),
        in_specs=[
            pl.BlockSpec((RB, d), lambda r: (r, 0)),
            pl.BlockSpec((8, d), lambda r: (0, 0)),
        ],
        out_specs=pl.BlockSpec((8, d), lambda r: (0, 0)),
        out_shape=jax.ShapeDtypeStruct((8, d), jnp.float32),
    )


def _t1(n, d_in, d_out, n_chunks):
    nb = n // RB

    def body(z_ref, st_ref, ss_ref, g_ref, bt_ref, w_ref, dinv_ref, *out_refs):
        mean = st_ref[0:1, :] * (1.0 / n)
        var = ss_ref[0:1, :] * (1.0 / n)
        inv = 1.0 / jnp.sqrt(var + 1e-5)
        h = (z_ref[...] - mean) * (inv * g_ref[...]) + bt_ref[...]
        h = jnp.where(h >= 0, h, 0.01 * h)
        xs = _dotT(h, w_ref) * dinv_ref[...]
        for oref, piece in zip(out_refs, _pad_chunks(xs, d_out, n_chunks)):
            oref[...] = piece

    return pl.pallas_call(
        body,
        grid=(nb,),
        in_specs=[
            pl.BlockSpec((RB, d_in), lambda r: (r, 0)),
            pl.BlockSpec((8, d_in), lambda r: (0, 0)),
            pl.BlockSpec((8, d_in), lambda r: (0, 0)),
            pl.BlockSpec((1, d_in), lambda r: (0, 0)),
            pl.BlockSpec((1, d_in), lambda r: (0, 0)),
            pl.BlockSpec((d_out, d_in), lambda r: (0, 0)),
            pl.BlockSpec((RB, 1), lambda r: (r, 0)),
        ],
        out_specs=[pl.BlockSpec((RB, W), lambda r: (r, 0))
                   for _ in range(n_chunks)],
        out_shape=[jax.ShapeDtypeStruct((n, W), jnp.float32)
                   for _ in range(n_chunks)],
    )


def _t2(n, d, n_chunks, n_planes):
    nb = n // RB
    split = n_chunks == 1

    def body(accp_ref, *rest):
        xs_refs = rest[:n_chunks]
        dinv_ref, b_ref, z_ref, st_ref = rest[n_chunks:]
        parts = []
        for c in range(n_chunks):
            a = (accp_ref[0] + accp_ref[1]) if split else accp_ref[c]
            wc = min((c + 1) * W, d) - c * W
            parts.append((a + xs_refs[c][...])[:, :wc])
        agg = jnp.concatenate(parts, axis=1) if n_chunks > 1 else parts[0]
        z = agg * dinv_ref[...] + b_ref[...]
        z_ref[...] = z
        ps = jnp.concatenate(
            [jnp.sum(z, axis=0, keepdims=True),
             jnp.sum(z * z, axis=0, keepdims=True),
             jnp.zeros((6, d), jnp.float32)], axis=0)
        r = pl.program_id(0)

        @pl.when(r == 0)
        def _():
            st_ref[...] = ps

        @pl.when(r > 0)
        def _():
            st_ref[...] = st_ref[...] + ps

    return pl.pallas_call(
        body,
        grid=(nb,),
        in_specs=[pl.BlockSpec((n_planes, RB, W), lambda r: (0, r, 0))]
        + [pl.BlockSpec((RB, W), lambda r: (r, 0)) for _ in range(n_chunks)]
        + [
            pl.BlockSpec((RB, 1), lambda r: (r, 0)),
            pl.BlockSpec((1, d), lambda r: (0, 0)),
        ],
        out_specs=[
            pl.BlockSpec((RB, d), lambda r: (r, 0)),
            pl.BlockSpec((8, d), lambda r: (0, 0)),
        ],
        out_shape=[
            jax.ShapeDtypeStruct((n, d), jnp.float32),
            jax.ShapeDtypeStruct((8, d), jnp.float32),
        ],
    )


def _head(n, d, d_out):
    nb = n // RB

    def body(z_ref, st_ref, ss_ref, g_ref, bt_ref, lw_ref, lb_ref, o_ref):
        mean = st_ref[0:1, :] * (1.0 / n)
        var = ss_ref[0:1, :] * (1.0 / n)
        inv = 1.0 / jnp.sqrt(var + 1e-5)
        h = (z_ref[...] - mean) * (inv * g_ref[...]) + bt_ref[...]
        h = jnp.where(h >= 0, h, 0.01 * h)
        t = jnp.tanh(_dotT(h, lw_ref) + lb_ref[...])
        s = jnp.sum(t * t, axis=1, keepdims=True)
        o_ref[...] = t / (jnp.sqrt(s) + 1e-12)

    return pl.pallas_call(
        body,
        grid=(nb,),
        in_specs=[
            pl.BlockSpec((RB, d), lambda r: (r, 0)),
            pl.BlockSpec((8, d), lambda r: (0, 0)),
            pl.BlockSpec((8, d), lambda r: (0, 0)),
            pl.BlockSpec((1, d), lambda r: (0, 0)),
            pl.BlockSpec((1, d), lambda r: (0, 0)),
            pl.BlockSpec((d_out, d), lambda r: (0, 0)),
            pl.BlockSpec((1, d_out), lambda r: (0, 0)),
        ],
        out_specs=pl.BlockSpec((RB, d_out), lambda r: (r, 0)),
        out_shape=jax.ShapeDtypeStruct((n, d_out), jnp.float32),
    )


def kernel(x, edge_index, Ws, bs, gammas, betas, linW, linb):
    n = x.shape[0]
    e = edge_index.shape[1]
    n_layers = len(Ws)
    src = edge_index[0]
    dst = edge_index[1]

    e_rows = -(-e // (K * 64)) * 64
    ep = e_rows * K
    pad = ep - e
    pad_idx = jnp.arange(pad, dtype=jnp.int32)
    src_p = jnp.concatenate([src, (pad_idx * 97) % n])
    dst_p = jnp.concatenate([dst, n + (pad_idx % 64)])
    src2 = src_p.reshape(e_rows, K)
    dst2 = dst_p.reshape(e_rows, K)

    zeros_w = jnp.zeros((SROWS, W), jnp.float32)

    deg_p = _make_deg(e_rows)(
        dst2, jnp.ones((K, W), jnp.float32), zeros_w)

    dinv = None
    z = None
    stats = None
    ssq = None
    for i in range(n_layers):
        d_in = Ws[i].shape[1]
        d = Ws[i].shape[0]
        n_chunks = -(-d // W)
        n_planes = NC if n_chunks == 1 else n_chunks

        if i == 0:
            *xs_chunks, dinv = _t1_first(n, d_in, d, n_chunks)(
                x, deg_p, Ws[i])
        else:
            xs_chunks = list(_t1(n, d_in, d, n_chunks)(
                z, stats, ssq, gammas[i - 1].reshape(1, -1),
                betas[i - 1].reshape(1, -1), Ws[i], dinv))

        acc_p = _make_agg(e_rows, n_chunks)(
            src2, dst2, zeros_w, *xs_chunks)
        z, stats = _t2(n, d, n_chunks, n_planes)(
            acc_p, *xs_chunks, dinv, bs[i].reshape(1, -1))
        ssq = _t3(n, d)(z, stats)

    out = _head(n, Ws[-1].shape[0], linW.shape[0])(
        z, stats, ssq, gammas[-1].reshape(1, -1), betas[-1].reshape(1, -1),
        linW, linb.reshape(1, -1))
    return out
